# baseline (speedup 1.0000x reference)
import jax
import jax.numpy as jnp
from jax import lax
from jax.experimental import pallas as pl
from jax.experimental.pallas import tpu as pltpu
from jax.experimental.pallas import tpu_sc as plsc

BATCH = 16384
D = 32
N_NEGS = 20
NC = 2
NS = 16
NW = NC * NS
BPW = BATCH // NW
CB = 32
NCHUNK = BPW // CB
ROWS = CB * N_NEGS
GPC = ROWS // 16
L = 16


def _dot16(h_ref, a_ref, b_ref, hrow, srow):
    acc = jnp.zeros((L,), jnp.float32)
    for d in range(D):
        dc = jnp.full((L,), d, jnp.int32)
        hv = plsc.load_gather(h_ref, [hrow, dc])
        av = plsc.load_gather(a_ref, [srow, dc])
        bv = plsc.load_gather(b_ref, [srow, dc])
        acc = acc + hv * (av + bv)
    return acc


def _body(users_r, itemsi_r, itemsj_r, negs_r, h_r, p_r, q_r,
          outpos_r, outneg_r,
          idx_u, idx_i, idx_j, idx_n,
          hbuf, pbuf, qbuf, npbuf, nqbuf, posbuf, negbuf, sem):
    wid = lax.axis_index("s") * NC + lax.axis_index("c")

    pltpu.sync_copy(users_r.at[pl.ds(wid * 4, 4)], idx_u)
    pltpu.sync_copy(itemsi_r.at[pl.ds(wid * 4, 4)], idx_i)
    pltpu.sync_copy(itemsj_r.at[pl.ds(wid * 4, 4)], idx_j)
    pltpu.sync_copy(negs_r.at[pl.ds(wid * 80, 80)], idx_n)

    handles = []
    for k in range(4):
        dst = pl.ds(k * 128, 128)
        handles.append(pltpu.async_copy(h_r.at[idx_u.at[k]], hbuf.at[dst], sem))
        handles.append(pltpu.async_copy(p_r.at[idx_i.at[k]], pbuf.at[dst], sem))
        handles.append(pltpu.async_copy(q_r.at[idx_j.at[k]], qbuf.at[dst], sem))
    for h in handles:
        h.wait()

    def pos_group(g, carry):
        flat = jnp.full((L,), g * L, jnp.int32) + lax.iota(jnp.int32, L)
        acc = _dot16(hbuf, pbuf, qbuf, flat, flat)
        posbuf[pl.ds(pl.multiple_of(g * L, L), L)] = acc
        return carry

    lax.fori_loop(0, BPW // L, pos_group, 0)
    pltpu.sync_copy(posbuf, outpos_r.at[pl.ds(wid * BPW, BPW)])

    def neg_chunk(c, carry):
        hs = []
        for k in range(5):
            dst = pl.ds(k * 128, 128)
            row = idx_n.at[c * 5 + k]
            hs.append(pltpu.async_copy(p_r.at[row], npbuf.at[dst], sem))
            hs.append(pltpu.async_copy(q_r.at[row], nqbuf.at[dst], sem))
        for h in hs:
            h.wait()

        def neg_group(g, carry2):
            flat = jnp.full((L,), g * L, jnp.int32) + lax.iota(jnp.int32, L)
            b_loc = flat // N_NEGS
            hrow = jnp.full((L,), c * CB, jnp.int32) + b_loc
            acc = _dot16(hbuf, npbuf, nqbuf, hrow, flat)
            negbuf[pl.ds(pl.multiple_of(g * L, L), L)] = acc
            return carry2

        lax.fori_loop(0, GPC, neg_group, 0)
        base = pl.multiple_of(wid * (BPW * N_NEGS) + c * ROWS, 8)
        pltpu.sync_copy(negbuf, outneg_r.at[pl.ds(base, ROWS)])
        return carry

    lax.fori_loop(0, NCHUNK, neg_chunk, 0)


@jax.jit
def _run(users2d, itemsi2d, itemsj2d, negs2d, H, P, Q):
    mesh = plsc.VectorSubcoreMesh(core_axis_name="c", subcore_axis_name="s",
                                  num_cores=NC, num_subcores=NS)
    f = pl.kernel(
        _body,
        out_type=[
            jax.ShapeDtypeStruct((BATCH,), jnp.float32),
            jax.ShapeDtypeStruct((BATCH * N_NEGS,), jnp.float32),
        ],
        mesh=mesh,
        compiler_params=pltpu.CompilerParams(needs_layout_passes=False,
                                             use_tc_tiling_on_sc=False),
        scratch_types=[
            pltpu.VMEM((4, 128), jnp.int32),
            pltpu.VMEM((4, 128), jnp.int32),
            pltpu.VMEM((4, 128), jnp.int32),
            pltpu.VMEM((80, 128), jnp.int32),
            pltpu.VMEM((BPW, D), jnp.float32),
            pltpu.VMEM((BPW, D), jnp.float32),
            pltpu.VMEM((BPW, D), jnp.float32),
            pltpu.VMEM((ROWS, D), jnp.float32),
            pltpu.VMEM((ROWS, D), jnp.float32),
            pltpu.VMEM((BPW,), jnp.float32),
            pltpu.VMEM((ROWS,), jnp.float32),
            pltpu.SemaphoreType.DMA,
        ],
    )
    return f(users2d, itemsi2d, itemsj2d, negs2d, H, P, Q)


def kernel(users, items_i, items_j, negs, H, P, Q):
    users2d = users.reshape(BATCH // 128, 128)
    itemsi2d = items_i.reshape(BATCH // 128, 128)
    itemsj2d = items_j.reshape(BATCH // 128, 128)
    negs2d = negs.reshape(BATCH * N_NEGS // 128, 128)
    pos, neg = _run(users2d, itemsi2d, itemsj2d, negs2d, H, P, Q)
    return pos, neg.reshape(BATCH, N_NEGS)


# speedup vs baseline: 1.2717x; 1.2717x over previous
import jax
import jax.numpy as jnp
from jax import lax
from jax.experimental import pallas as pl
from jax.experimental.pallas import tpu as pltpu
from jax.experimental.pallas import tpu_sc as plsc

BATCH = 16384
D = 32
N_NEGS = 20
NC = 2
NS = 16
NW = NC * NS
BPW = BATCH // NW
CB = 16
NCHUNK = BPW // CB
ROWS = CB * N_NEGS
GPC = ROWS // 16
L = 16


def _dot16(h_ref, a_ref, b_ref, hrow, srow):
    acc = jnp.zeros((L,), jnp.float32)
    for d in range(D):
        dc = jnp.full((L,), d, jnp.int32)
        hv = plsc.load_gather(h_ref, [hrow, dc])
        av = plsc.load_gather(a_ref, [srow, dc])
        bv = plsc.load_gather(b_ref, [srow, dc])
        acc = acc + hv * (av + bv)
    return acc


def _body(hrows_r, itemsi_r, itemsj_r, negs_r, p_r, q_r,
          outpos_r, outneg_r,
          idx_i, idx_j, idx_n,
          hbuf, pbuf, qbuf, npA, nqA, npB, nqB, posbuf, negbuf,
          semP, semA, semB):
    wid = lax.axis_index("s") * NC + lax.axis_index("c")
    nbase = wid * (BPW * N_NEGS)

    pltpu.sync_copy(itemsi_r.at[pl.ds(wid * BPW, BPW)], idx_i)
    pltpu.sync_copy(itemsj_r.at[pl.ds(wid * BPW, BPW)], idx_j)
    pltpu.sync_copy(negs_r.at[pl.ds(nbase, BPW * N_NEGS)], idx_n)

    pltpu.async_copy(hrows_r.at[pl.ds(wid * BPW, BPW)], hbuf, semP)
    for k in range(BPW // 128):
        dst = pl.ds(k * 128, 128)
        src = pl.ds(k * 128, 128)
        pltpu.async_copy(p_r.at[idx_i.at[src]], pbuf.at[dst], semP)
        pltpu.async_copy(q_r.at[idx_j.at[src]], qbuf.at[dst], semP)

    def fire(c, np_buf, nq_buf, sem):
        base = c * ROWS
        for off, n in ((0, 128), (128, 128), (256, 64)):
            row = pl.ds(base + off, n)
            dst = pl.ds(off, n)
            pltpu.async_copy(p_r.at[idx_n.at[row]], np_buf.at[dst], sem)
            pltpu.async_copy(q_r.at[idx_n.at[row]], nq_buf.at[dst], sem)

    def drain(np_buf, nq_buf, sem):
        pltpu.make_async_copy(p_r.at[pl.ds(0, ROWS)], np_buf, sem).wait()
        pltpu.make_async_copy(q_r.at[pl.ds(0, ROWS)], nq_buf, sem).wait()

    fire(0, npA, nqA, semA)

    pltpu.make_async_copy(hrows_r.at[pl.ds(0, BPW)], hbuf, semP).wait()
    pltpu.make_async_copy(p_r.at[pl.ds(0, BPW)], pbuf, semP).wait()
    pltpu.make_async_copy(q_r.at[pl.ds(0, BPW)], qbuf, semP).wait()

    def pos_group(g, carry):
        flat = jnp.full((L,), g * L, jnp.int32) + lax.iota(jnp.int32, L)
        acc = _dot16(hbuf, pbuf, qbuf, flat, flat)
        posbuf[pl.ds(pl.multiple_of(g * L, L), L)] = acc
        return carry

    lax.fori_loop(0, BPW // L, pos_group, 0)
    pltpu.sync_copy(posbuf, outpos_r.at[pl.ds(wid * BPW, BPW)])

    def compute_chunk(c, np_buf, nq_buf):
        def neg_group(g, carry):
            flat = jnp.full((L,), g * L, jnp.int32) + lax.iota(jnp.int32, L)
            b_loc = flat // N_NEGS
            hrow = jnp.full((L,), c * CB, jnp.int32) + b_loc
            acc = _dot16(hbuf, np_buf, nq_buf, hrow, flat)
            negbuf[pl.ds(pl.multiple_of(g * L, L), L)] = acc
            return carry

        lax.fori_loop(0, GPC, neg_group, 0)
        base = pl.multiple_of(nbase + c * ROWS, 8)
        pltpu.sync_copy(negbuf, outneg_r.at[pl.ds(base, ROWS)])

    def pair(t, carry):
        c0 = t * 2
        fire(c0 + 1, npB, nqB, semB)
        drain(npA, nqA, semA)
        compute_chunk(c0, npA, nqA)

        @pl.when(t + 1 < NCHUNK // 2)
        def _():
            fire(c0 + 2, npA, nqA, semA)

        drain(npB, nqB, semB)
        compute_chunk(c0 + 1, npB, nqB)
        return carry

    lax.fori_loop(0, NCHUNK // 2, pair, 0)


@jax.jit
def _run(hrows, itemsi, itemsj, negs, P, Q):
    mesh = plsc.VectorSubcoreMesh(core_axis_name="c", subcore_axis_name="s",
                                  num_cores=NC, num_subcores=NS)
    f = pl.kernel(
        _body,
        out_type=[
            jax.ShapeDtypeStruct((BATCH,), jnp.float32),
            jax.ShapeDtypeStruct((BATCH * N_NEGS,), jnp.float32),
        ],
        mesh=mesh,
        compiler_params=pltpu.CompilerParams(needs_layout_passes=False,
                                             use_tc_tiling_on_sc=False),
        scratch_types=[
            pltpu.VMEM((BPW,), jnp.int32),
            pltpu.VMEM((BPW,), jnp.int32),
            pltpu.VMEM((BPW * N_NEGS,), jnp.int32),
            pltpu.VMEM((BPW, D), jnp.float32),
            pltpu.VMEM((BPW, D), jnp.float32),
            pltpu.VMEM((BPW, D), jnp.float32),
            pltpu.VMEM((ROWS, D), jnp.float32),
            pltpu.VMEM((ROWS, D), jnp.float32),
            pltpu.VMEM((ROWS, D), jnp.float32),
            pltpu.VMEM((ROWS, D), jnp.float32),
            pltpu.VMEM((BPW,), jnp.float32),
            pltpu.VMEM((ROWS,), jnp.float32),
            pltpu.SemaphoreType.DMA,
            pltpu.SemaphoreType.DMA,
            pltpu.SemaphoreType.DMA,
        ],
    )
    return f(hrows, itemsi, itemsj, negs, P, Q)


def kernel(users, items_i, items_j, negs, H, P, Q):
    h_u = jnp.take(H, users, axis=0)
    pos, neg = _run(h_u, items_i, items_j, negs.reshape(-1), P, Q)
    return pos, neg.reshape(BATCH, N_NEGS)


# speedup vs baseline: 1.3554x; 1.0658x over previous
import jax
import jax.numpy as jnp
from jax import lax
from jax.experimental import pallas as pl
from jax.experimental.pallas import tpu as pltpu
from jax.experimental.pallas import tpu_sc as plsc

BATCH = 16384
D = 32
N_NEGS = 20
NC = 2
NS = 16
NW = NC * NS
BPW = BATCH // NW
CB = 16
NCHUNK = BPW // CB
ROWS = CB * N_NEGS
GPC = ROWS // 16
L = 16


def _dot16(h_ref, a_ref, b_ref, hrow, srow):
    acc = jnp.zeros((L,), jnp.float32)
    for d in range(D):
        dc = jnp.full((L,), d, jnp.int32)
        hv = plsc.load_gather(h_ref, [hrow, dc])
        av = plsc.load_gather(a_ref, [srow, dc])
        bv = plsc.load_gather(b_ref, [srow, dc])
        acc = acc + hv * (av + bv)
    return acc


def _body(hrows_r, itemsi_r, itemsj_r, negs_r, p_r, q_r,
          outpos_r, outneg_r,
          idx_i, idx_j, idx_n,
          hbuf, pbuf, qbuf, npA, nqA, npB, nqB, posbuf, negbuf,
          semP, semA, semB):
    wid = lax.axis_index("s") * NC + lax.axis_index("c")
    nbase = wid * (BPW * N_NEGS)

    pltpu.sync_copy(itemsi_r.at[pl.ds(wid * BPW, BPW)], idx_i)
    pltpu.sync_copy(itemsj_r.at[pl.ds(wid * BPW, BPW)], idx_j)
    pltpu.sync_copy(negs_r.at[pl.ds(nbase, BPW * N_NEGS)], idx_n)

    pltpu.async_copy(hrows_r.at[pl.ds(wid * BPW, BPW)], hbuf, semP)
    for k in range(BPW // 128):
        dst = pl.ds(k * 128, 128)
        src = pl.ds(k * 128, 128)
        pltpu.async_copy(p_r.at[idx_i.at[src]], pbuf.at[dst], semP)
        pltpu.async_copy(q_r.at[idx_j.at[src]], qbuf.at[dst], semP)

    def fire(c, np_buf, nq_buf, sem):
        base = c * ROWS
        for off, n in ((0, 128), (128, 128), (256, 64)):
            row = pl.ds(base + off, n)
            dst = pl.ds(off, n)
            pltpu.async_copy(p_r.at[idx_n.at[row]], np_buf.at[dst], sem)
            pltpu.async_copy(q_r.at[idx_n.at[row]], nq_buf.at[dst], sem)

    def drain(np_buf, nq_buf, sem):
        pltpu.make_async_copy(p_r.at[pl.ds(0, ROWS)], np_buf, sem).wait()
        pltpu.make_async_copy(q_r.at[pl.ds(0, ROWS)], nq_buf, sem).wait()

    fire(0, npA, nqA, semA)

    pltpu.make_async_copy(hrows_r.at[pl.ds(0, BPW)], hbuf, semP).wait()
    pltpu.make_async_copy(p_r.at[pl.ds(0, BPW)], pbuf, semP).wait()
    pltpu.make_async_copy(q_r.at[pl.ds(0, BPW)], qbuf, semP).wait()

    def pos_group(g, carry):
        flat = jnp.full((L,), g * L, jnp.int32) + lax.iota(jnp.int32, L)
        acc = _dot16(hbuf, pbuf, qbuf, flat, flat)
        posbuf[pl.ds(pl.multiple_of(g * L, L), L)] = acc
        return carry

    lax.fori_loop(0, BPW // L, pos_group, 0)
    pltpu.sync_copy(posbuf, outpos_r.at[pl.ds(wid * BPW, BPW)])

    def compute_chunk(c, np_buf, nq_buf):
        def neg_block(g5, carry):
            base_b = c * CB + g5 * 4
            for r in range(5):
                lanes = [(r * L + l) // N_NEGS for l in range(L)]
                b0, b1 = lanes[0], lanes[-1]
                n0 = lanes.count(b0)
                mask = lax.iota(jnp.int32, L) < n0
                h0 = [hbuf[base_b + b0, pl.ds(pl.multiple_of(k * L, L), L)]
                      for k in range(D // L)]
                if b1 != b0:
                    h1 = [hbuf[base_b + b1, pl.ds(pl.multiple_of(k * L, L), L)]
                          for k in range(D // L)]
                flat = (jnp.full((L,), (g5 * 5 + r) * L, jnp.int32)
                        + lax.iota(jnp.int32, L))
                acc = jnp.zeros((L,), jnp.float32)
                for d in range(D):
                    dc = jnp.full((L,), d, jnp.int32)
                    lane = jnp.full((L,), d % L, jnp.int32)
                    hv = jnp.take_along_axis(h0[d // L], lane, axis=0)
                    if b1 != b0:
                        hv1 = jnp.take_along_axis(h1[d // L], lane, axis=0)
                        hv = jnp.where(mask, hv, hv1)
                    av = plsc.load_gather(np_buf, [flat, dc])
                    bv = plsc.load_gather(nq_buf, [flat, dc])
                    acc = acc + hv * (av + bv)
                off = pl.multiple_of((g5 * 5 + r) * L, L)
                negbuf[pl.ds(off, L)] = acc
            return carry

        lax.fori_loop(0, GPC // 5, neg_block, 0)
        base = pl.multiple_of(nbase + c * ROWS, 8)
        pltpu.sync_copy(negbuf, outneg_r.at[pl.ds(base, ROWS)])

    def pair(t, carry):
        c0 = t * 2
        fire(c0 + 1, npB, nqB, semB)
        drain(npA, nqA, semA)
        compute_chunk(c0, npA, nqA)

        @pl.when(t + 1 < NCHUNK // 2)
        def _():
            fire(c0 + 2, npA, nqA, semA)

        drain(npB, nqB, semB)
        compute_chunk(c0 + 1, npB, nqB)
        return carry

    lax.fori_loop(0, NCHUNK // 2, pair, 0)


@jax.jit
def _run(hrows, itemsi, itemsj, negs, P, Q):
    mesh = plsc.VectorSubcoreMesh(core_axis_name="c", subcore_axis_name="s",
                                  num_cores=NC, num_subcores=NS)
    f = pl.kernel(
        _body,
        out_type=[
            jax.ShapeDtypeStruct((BATCH,), jnp.float32),
            jax.ShapeDtypeStruct((BATCH * N_NEGS,), jnp.float32),
        ],
        mesh=mesh,
        compiler_params=pltpu.CompilerParams(needs_layout_passes=False,
                                             use_tc_tiling_on_sc=False),
        scratch_types=[
            pltpu.VMEM((BPW,), jnp.int32),
            pltpu.VMEM((BPW,), jnp.int32),
            pltpu.VMEM((BPW * N_NEGS,), jnp.int32),
            pltpu.VMEM((BPW, D), jnp.float32),
            pltpu.VMEM((BPW, D), jnp.float32),
            pltpu.VMEM((BPW, D), jnp.float32),
            pltpu.VMEM((ROWS, D), jnp.float32),
            pltpu.VMEM((ROWS, D), jnp.float32),
            pltpu.VMEM((ROWS, D), jnp.float32),
            pltpu.VMEM((ROWS, D), jnp.float32),
            pltpu.VMEM((BPW,), jnp.float32),
            pltpu.VMEM((ROWS,), jnp.float32),
            pltpu.SemaphoreType.DMA,
            pltpu.SemaphoreType.DMA,
            pltpu.SemaphoreType.DMA,
        ],
    )
    return f(hrows, itemsi, itemsj, negs, P, Q)


def kernel(users, items_i, items_j, negs, H, P, Q):
    h_u = jnp.take(H, users, axis=0)
    pos, neg = _run(h_u, items_i, items_j, negs.reshape(-1), P, Q)
    return pos, neg.reshape(BATCH, N_NEGS)


# speedup vs baseline: 1.5891x; 1.1725x over previous
import functools

import jax
import jax.numpy as jnp
from jax import lax
from jax.experimental import pallas as pl
from jax.experimental.pallas import tpu as pltpu
from jax.experimental.pallas import tpu_sc as plsc

BATCH = 16384
D = 32
N_NEGS = 20
NC = 2
NS = 16
NW = NC * NS
BPW = BATCH // NW
CB = 16
NCHUNK = BPW // CB
ROWS = CB * N_NEGS
GPC = ROWS // 16
L = 16


def _neg_block(c, g5, hbuf, t_buf, negbuf, part_buf):
    base_b = c * CB + g5 * 4
    for r in range(5):
        lanes = [(r * L + l) // N_NEGS for l in range(L)]
        b0, b1 = lanes[0], lanes[-1]
        n0 = lanes.count(b0)
        mask = lax.iota(jnp.int32, L) < n0
        h0 = [hbuf[base_b + b0, pl.ds(pl.multiple_of(k * L, L), L)]
              for k in range(D // L)]
        if b1 != b0:
            h1 = [hbuf[base_b + b1, pl.ds(pl.multiple_of(k * L, L), L)]
                  for k in range(D // L)]
        flat = (jnp.full((L,), (g5 * 5 + r) * L, jnp.int32)
                + lax.iota(jnp.int32, L))
        off = pl.multiple_of((g5 * 5 + r) * L, L)
        if part_buf is None:
            acc = jnp.zeros((L,), jnp.float32)
        else:
            acc = part_buf[pl.ds(off, L)]
        for d in range(D):
            dc = jnp.full((L,), d, jnp.int32)
            lane = jnp.full((L,), d % L, jnp.int32)
            hv = jnp.take_along_axis(h0[d // L], lane, axis=0)
            if b1 != b0:
                hv1 = jnp.take_along_axis(h1[d // L], lane, axis=0)
                hv = jnp.where(mask, hv, hv1)
            av = plsc.load_gather(t_buf, [flat, dc])
            acc = acc + hv * av
        negbuf[pl.ds(off, L)] = acc


def _make_body(with_partial):
    def body(*refs):
        if with_partial:
            (hrows_r, items_r, negs_r, t_r, ppos_r, pneg_r,
             outpos_r, outneg_r,
             idx_t, idx_n, hbuf, tbuf,
             nA, nB, partA, partB, pposbuf, posbuf, negbuf,
             semP, semA, semB) = refs
        else:
            (hrows_r, items_r, negs_r, t_r,
             outpos_r, outneg_r,
             idx_t, idx_n, hbuf, tbuf,
             nA, nB, pposbuf, posbuf, negbuf,
             semP, semA, semB) = refs
            ppos_r = pneg_r = partA = partB = None

        wid = lax.axis_index("s") * NC + lax.axis_index("c")
        nbase = wid * (BPW * N_NEGS)

        pltpu.sync_copy(items_r.at[pl.ds(wid * BPW, BPW)], idx_t)
        pltpu.sync_copy(negs_r.at[pl.ds(nbase, BPW * N_NEGS)], idx_n)

        pltpu.async_copy(hrows_r.at[pl.ds(wid * BPW, BPW)], hbuf, semP)
        for k in range(BPW // 128):
            sl = pl.ds(k * 128, 128)
            pltpu.async_copy(t_r.at[idx_t.at[sl]], tbuf.at[sl], semP)
        if with_partial:
            pltpu.async_copy(ppos_r.at[pl.ds(wid * BPW, BPW)], pposbuf, semP)

        def fire(c, n_buf, part, sem):
            base = c * ROWS
            for off, n in ((0, 128), (128, 128), (256, 64)):
                row = pl.ds(base + off, n)
                pltpu.async_copy(t_r.at[idx_n.at[row]],
                                 n_buf.at[pl.ds(off, n)], sem)
            if with_partial:
                pltpu.async_copy(
                    pneg_r.at[pl.ds(pl.multiple_of(nbase + base, 8), ROWS)],
                    part, sem)

        def drain(c, n_buf, part, sem):
            pltpu.make_async_copy(t_r.at[pl.ds(0, ROWS)], n_buf, sem).wait()
            if with_partial:
                pltpu.make_async_copy(
                    pneg_r.at[pl.ds(0, ROWS)], part, sem).wait()

        fire(0, nA, partA, semA)

        pltpu.make_async_copy(hrows_r.at[pl.ds(0, BPW)], hbuf, semP).wait()
        pltpu.make_async_copy(t_r.at[pl.ds(0, BPW)], tbuf, semP).wait()
        if with_partial:
            pltpu.make_async_copy(
                ppos_r.at[pl.ds(0, BPW)], pposbuf, semP).wait()

        def pos_group(g, carry):
            flat = (jnp.full((L,), g * L, jnp.int32)
                    + lax.iota(jnp.int32, L))
            off = pl.multiple_of(g * L, L)
            if with_partial:
                acc = pposbuf[pl.ds(off, L)]
            else:
                acc = jnp.zeros((L,), jnp.float32)
            for d in range(D):
                dc = jnp.full((L,), d, jnp.int32)
                hv = plsc.load_gather(hbuf, [flat, dc])
                tv = plsc.load_gather(tbuf, [flat, dc])
                acc = acc + hv * tv
            posbuf[pl.ds(off, L)] = acc
            return carry

        lax.fori_loop(0, BPW // L, pos_group, 0)
        pltpu.sync_copy(posbuf, outpos_r.at[pl.ds(wid * BPW, BPW)])

        def compute_chunk(c, n_buf, part):
            def blk(g5, carry):
                _neg_block(c, g5, hbuf, n_buf, negbuf, part)
                return carry

            lax.fori_loop(0, GPC // 5, blk, 0)
            base = pl.multiple_of(nbase + c * ROWS, 8)
            pltpu.sync_copy(negbuf, outneg_r.at[pl.ds(base, ROWS)])

        def pair(t, carry):
            c0 = t * 2
            fire(c0 + 1, nB, partB, semB)
            drain(c0, nA, partA, semA)
            compute_chunk(c0, nA, partA)

            @pl.when(t + 1 < NCHUNK // 2)
            def _():
                fire(c0 + 2, nA, partA, semA)

            drain(c0 + 1, nB, partB, semB)
            compute_chunk(c0 + 1, nB, partB)
            return carry

        lax.fori_loop(0, NCHUNK // 2, pair, 0)

    return body


def _scratch(with_partial):
    s = [
        pltpu.VMEM((BPW,), jnp.int32),
        pltpu.VMEM((BPW * N_NEGS,), jnp.int32),
        pltpu.VMEM((BPW, D), jnp.float32),
        pltpu.VMEM((BPW, D), jnp.float32),
        pltpu.VMEM((ROWS, D), jnp.float32),
        pltpu.VMEM((ROWS, D), jnp.float32),
    ]
    if with_partial:
        s += [
            pltpu.VMEM((ROWS,), jnp.float32),
            pltpu.VMEM((ROWS,), jnp.float32),
        ]
    s += [
        pltpu.VMEM((BPW,), jnp.float32),
        pltpu.VMEM((BPW,), jnp.float32),
        pltpu.VMEM((ROWS,), jnp.float32),
        pltpu.SemaphoreType.DMA,
        pltpu.SemaphoreType.DMA,
        pltpu.SemaphoreType.DMA,
    ]
    return s


_OUT = [
    jax.ShapeDtypeStruct((BATCH,), jnp.float32),
    jax.ShapeDtypeStruct((BATCH * N_NEGS,), jnp.float32),
]
_PARAMS = pltpu.CompilerParams(needs_layout_passes=False,
                               use_tc_tiling_on_sc=False)


@jax.jit
def _run(hrows, itemsi, itemsj, negs, P, Q):
    mesh = plsc.VectorSubcoreMesh(core_axis_name="c", subcore_axis_name="s",
                                  num_cores=NC, num_subcores=NS)
    fp = pl.kernel(_make_body(False), out_type=_OUT, mesh=mesh,
                   compiler_params=_PARAMS, scratch_types=_scratch(False))
    fq = pl.kernel(_make_body(True), out_type=_OUT, mesh=mesh,
                   compiler_params=_PARAMS, scratch_types=_scratch(True))
    ppos, pneg = fp(hrows, itemsi, negs, P)
    return fq(hrows, itemsj, negs, Q, ppos, pneg)


def kernel(users, items_i, items_j, negs, H, P, Q):
    h_u = jnp.take(H, users, axis=0)
    pos, neg = _run(h_u, items_i, items_j, negs.reshape(-1), P, Q)
    return pos, neg.reshape(BATCH, N_NEGS)
